# baseline (speedup 1.0000x reference)
import dataclasses

import jax
import jax.numpy as jnp
from jax import lax
from jax.experimental import pallas as pl
from jax.experimental.pallas import tpu as pltpu
from jax.experimental.pallas import tpu_sc as plsc

N_NODES = 10000
N_EDGES = 320000
D = 128

NC = 2
NS = 16
NW = NC * NS
CHUNK = 128
CPW = 80
N_PAD = NW * CPW * CHUNK - N_EDGES

ROWS_PER_SUB = 624
TAIL_ROWS = N_NODES - NS * ROWS_PER_SUB
ZROWS = 104


def _matmul_body(x_ref, w_ref, o_ref):
    o_ref[...] = jnp.dot(x_ref[...], w_ref[...],
                         preferred_element_type=jnp.float32)


def _matmul(x, W):
    blk = 1000
    return pl.pallas_call(
        _matmul_body,
        grid=(N_NODES // blk,),
        in_specs=[
            pl.BlockSpec((blk, D), lambda i: (i, 0)),
            pl.BlockSpec((D, D), lambda i: (0, 0)),
        ],
        out_specs=pl.BlockSpec((blk, D), lambda i: (i, 0)),
        out_shape=jax.ShapeDtypeStruct((N_NODES, D), jnp.float32),
    )(x, W)


def _combine_body(p_ref, o_ref):
    o_ref[...] = jnp.maximum(p_ref[0] + p_ref[1], 0.0)


def _combine(partials):
    blk = 1000
    return pl.pallas_call(
        _combine_body,
        grid=(N_NODES // blk,),
        in_specs=[pl.BlockSpec((2, blk, D), lambda i: (0, i, 0))],
        out_specs=pl.BlockSpec((blk, D), lambda i: (i, 0)),
        out_shape=jax.ShapeDtypeStruct((N_NODES, D), jnp.float32),
    )(partials)


def _sc_body(support_hbm, meta_hbm, out_hbm,
             acc_spmem,
             metaA, rowsA, semA,
             metaB, rowsB, semB):
    core = lax.axis_index("c")
    sub = lax.axis_index("s")
    wid = sub * NC + core

    @pl.loop(0, ZROWS)
    def _(r):
        for g in range(D // 16):
            rowsA[r, pl.ds(g * 16, 16)] = jnp.zeros((16,), jnp.float32)

    base = sub * ROWS_PER_SUB
    for k in range(ROWS_PER_SUB // ZROWS):
        pltpu.sync_copy(rowsA.at[pl.ds(0, ZROWS)],
                        acc_spmem.at[pl.ds(base + k * ZROWS, ZROWS)])

    @pl.when(sub == NS - 1)
    def _():
        pltpu.sync_copy(rowsA.at[pl.ds(0, TAIL_ROWS)],
                        acc_spmem.at[pl.ds(NS * ROWS_PER_SUB, TAIL_ROWS)])

    plsc.subcore_barrier()

    def load_and_gather(c, meta_v, rows, sem):
        pltpu.sync_copy(meta_hbm.at[c], meta_v)
        pltpu.async_copy(support_hbm.at[meta_v.at[0]], rows, sem)

    def finish(meta_v, rows, sem):
        pltpu.make_async_copy(support_hbm.at[meta_v.at[0]], rows, sem).wait()

        dnums = lax.GatherDimensionNumbers(
            offset_dims=(), collapsed_slice_dims=(0,), start_index_map=(0,))

        @plsc.parallel_loop(0, CHUNK, step=16, unroll=2)
        def _(eg):
            w16 = plsc.bitcast(meta_v[2, pl.ds(eg, 16)], jnp.float32)
            for j in range(16):
                jidx = jnp.full((16, 1), j, jnp.int32)
                bw = lax.gather(
                    w16, jidx, dnums, (1,),
                    mode=lax.GatherScatterMode.PROMISE_IN_BOUNDS)
                for g in range(D // 16):
                    sl = pl.ds(g * 16, 16)
                    rows[eg + j, sl] = rows[eg + j, sl] * bw

        pltpu.sync_copy(rows, acc_spmem.at[meta_v.at[1]], add=True)

    load_and_gather(wid, metaA, rowsA, semA)
    load_and_gather(wid + NW, metaB, rowsB, semB)

    @pl.loop(0, CPW, step=2)
    def _(k):
        c = wid + k * NW
        finish(metaA, rowsA, semA)

        @pl.when(k + 2 < CPW)
        def _():
            load_and_gather(c + 2 * NW, metaA, rowsA, semA)

        finish(metaB, rowsB, semB)

        @pl.when(k + 3 < CPW)
        def _():
            load_and_gather(c + 3 * NW, metaB, rowsB, semB)

    plsc.subcore_barrier()

    ob = core * N_NODES + base
    for k in range(ROWS_PER_SUB // ZROWS):
        pltpu.sync_copy(acc_spmem.at[pl.ds(base + k * ZROWS, ZROWS)],
                        out_hbm.at[pl.ds(ob + k * ZROWS, ZROWS)])

    @pl.when(sub == NS - 1)
    def _():
        pltpu.sync_copy(acc_spmem.at[pl.ds(NS * ROWS_PER_SUB, TAIL_ROWS)],
                        out_hbm.at[pl.ds(core * N_NODES + NS * ROWS_PER_SUB,
                                         TAIL_ROWS)])


def _sc_spmm(support, meta):
    mesh = plsc.VectorSubcoreMesh(core_axis_name="c", subcore_axis_name="s")
    cp = pltpu.CompilerParams()
    if "needs_layout_passes" in pltpu.CompilerParams.__dataclass_fields__:
        cp = dataclasses.replace(cp, needs_layout_passes=False)
    f = pl.kernel(
        _sc_body,
        compiler_params=cp,
        out_type=jax.ShapeDtypeStruct((NC * N_NODES, D), jnp.float32),
        mesh=mesh,
        scratch_types=[
            pltpu.VMEM_SHARED((N_NODES, D), jnp.float32),
            pltpu.VMEM((8, CHUNK), jnp.int32),
            pltpu.VMEM((CHUNK, D), jnp.float32),
            pltpu.SemaphoreType.DMA,
            pltpu.VMEM((8, CHUNK), jnp.int32),
            pltpu.VMEM((CHUNK, D), jnp.float32),
            pltpu.SemaphoreType.DMA,
        ],
    )
    return f(support, meta)


def kernel(x, edge_index, edge_weight, W):
    support = _matmul(x, W)
    pad_idx = jnp.arange(N_PAD, dtype=jnp.int32) % N_NODES
    dst = jnp.concatenate([edge_index[0], pad_idx]).reshape(-1, CHUNK)
    src = jnp.concatenate([edge_index[1], pad_idx]).reshape(-1, CHUNK)
    ew = jax.lax.bitcast_convert_type(
        jnp.pad(edge_weight, (0, N_PAD)), jnp.int32).reshape(-1, CHUNK)
    zpad = jnp.zeros((NW * CPW, 5, CHUNK), jnp.int32)
    meta = jnp.concatenate(
        [jnp.stack([src, dst, ew], axis=1), zpad], axis=1)
    partials = _sc_spmm(support, meta)
    return _combine(partials.reshape(NC, N_NODES, D))


# speedup vs baseline: 1.1320x; 1.1320x over previous
import dataclasses

import jax
import jax.numpy as jnp
from jax import lax
from jax.experimental import pallas as pl
from jax.experimental.pallas import tpu as pltpu
from jax.experimental.pallas import tpu_sc as plsc

N_NODES = 10000
N_EDGES = 320000
D = 128

NC = 2
NS = 16
NW = NC * NS
CHUNK = 112
CPW = 90
N_PAD = NW * CPW * CHUNK - N_EDGES

ROWS_PER_SUB = 624
TAIL_ROWS = N_NODES - NS * ROWS_PER_SUB
ZROWS = 104


def _matmul_body(x_ref, w_ref, o_ref):
    o_ref[...] = jnp.dot(x_ref[...], w_ref[...],
                         preferred_element_type=jnp.float32)


def _matmul(x, W):
    blk = 1000
    return pl.pallas_call(
        _matmul_body,
        grid=(N_NODES // blk,),
        in_specs=[
            pl.BlockSpec((blk, D), lambda i: (i, 0)),
            pl.BlockSpec((D, D), lambda i: (0, 0)),
        ],
        out_specs=pl.BlockSpec((blk, D), lambda i: (i, 0)),
        out_shape=jax.ShapeDtypeStruct((N_NODES, D), jnp.float32),
    )(x, W)


def _combine_body(p_ref, o_ref):
    o_ref[...] = jnp.maximum(p_ref[0] + p_ref[1], 0.0)


def _combine(partials):
    blk = 1000
    return pl.pallas_call(
        _combine_body,
        grid=(N_NODES // blk,),
        in_specs=[pl.BlockSpec((2, blk, D), lambda i: (0, i, 0))],
        out_specs=pl.BlockSpec((blk, D), lambda i: (i, 0)),
        out_shape=jax.ShapeDtypeStruct((N_NODES, D), jnp.float32),
    )(partials)


def _sc_body(support_hbm, meta_hbm, out_hbm,
             acc_spmem,
             meta0, rows0, semG0, semS0,
             meta1, rows1, semG1, semS1,
             meta2, rows2, semG2, semS2):
    core = lax.axis_index("c")
    sub = lax.axis_index("s")
    wid = sub * NC + core
    metas = (meta0, meta1, meta2)
    rowss = (rows0, rows1, rows2)
    semGs = (semG0, semG1, semG2)
    semSs = (semS0, semS1, semS2)
    rowsA = rows0

    @pl.loop(0, ZROWS)
    def _(r):
        for g in range(D // 16):
            rowsA[r, pl.ds(g * 16, 16)] = jnp.zeros((16,), jnp.float32)

    base = sub * ROWS_PER_SUB
    for k in range(ROWS_PER_SUB // ZROWS):
        pltpu.sync_copy(rowsA.at[pl.ds(0, ZROWS)],
                        acc_spmem.at[pl.ds(base + k * ZROWS, ZROWS)])

    @pl.when(sub == NS - 1)
    def _():
        pltpu.sync_copy(rowsA.at[pl.ds(0, TAIL_ROWS)],
                        acc_spmem.at[pl.ds(NS * ROWS_PER_SUB, TAIL_ROWS)])

    plsc.subcore_barrier()

    def refill(n, s):
        c = wid + n * NW
        pltpu.sync_copy(meta_hbm.at[c], metas[s])
        pltpu.async_copy(support_hbm.at[metas[s].at[0]], rowss[s], semGs[s])

    def process(s):
        meta_v, rows = metas[s], rowss[s]
        pltpu.make_async_copy(support_hbm.at[meta_v.at[0]], rows,
                              semGs[s]).wait()

        @plsc.parallel_loop(0, CHUNK, step=16, unroll=2)
        def _(eg):
            w16 = plsc.bitcast(meta_v[2, pl.ds(eg, 16)], jnp.float32)
            for j in range(16):
                bw = jnp.full((16,), w16[j], jnp.float32)
                for g in range(D // 16):
                    sl = pl.ds(g * 16, 16)
                    rows[eg + j, sl] = rows[eg + j, sl] * bw

        pltpu.async_copy(rows, acc_spmem.at[meta_v.at[1]], semSs[s],
                         add=True)

    def wait_scatter(s):
        pltpu.make_async_copy(rowss[s], acc_spmem.at[metas[s].at[1]],
                              semSs[s]).wait()

    refill(0, 0)
    refill(1, 1)

    @pl.loop(0, CPW, step=3)
    def _(k):
        for b in range(3):
            s = b % 3
            process(s)
            if b == 0:
                @pl.when(k > 0)
                def _():
                    wait_scatter((b + 2) % 3)
            else:
                wait_scatter((b + 2) % 3)

            @pl.when(k + b + 2 < CPW)
            def _():
                refill(k + b + 2, (b + 2) % 3)

    wait_scatter((CPW - 1) % 3)

    plsc.subcore_barrier()

    ob = core * N_NODES + base
    for k in range(ROWS_PER_SUB // ZROWS):
        pltpu.sync_copy(acc_spmem.at[pl.ds(base + k * ZROWS, ZROWS)],
                        out_hbm.at[pl.ds(ob + k * ZROWS, ZROWS)])

    @pl.when(sub == NS - 1)
    def _():
        pltpu.sync_copy(acc_spmem.at[pl.ds(NS * ROWS_PER_SUB, TAIL_ROWS)],
                        out_hbm.at[pl.ds(core * N_NODES + NS * ROWS_PER_SUB,
                                         TAIL_ROWS)])


def _sc_spmm(support, meta):
    mesh = plsc.VectorSubcoreMesh(core_axis_name="c", subcore_axis_name="s")
    cp = pltpu.CompilerParams()
    if "needs_layout_passes" in pltpu.CompilerParams.__dataclass_fields__:
        cp = dataclasses.replace(cp, needs_layout_passes=False)
    f = pl.kernel(
        _sc_body,
        compiler_params=cp,
        out_type=jax.ShapeDtypeStruct((NC * N_NODES, D), jnp.float32),
        mesh=mesh,
        scratch_types=[
            pltpu.VMEM_SHARED((N_NODES, D), jnp.float32),
        ] + [
            t
            for _ in range(3)
            for t in (pltpu.VMEM((8, CHUNK), jnp.int32),
                      pltpu.VMEM((CHUNK, D), jnp.float32),
                      pltpu.SemaphoreType.DMA,
                      pltpu.SemaphoreType.DMA)
        ],
    )
    return f(support, meta)


def kernel(x, edge_index, edge_weight, W):
    support = _matmul(x, W)
    pad_idx = jnp.arange(N_PAD, dtype=jnp.int32) % N_NODES
    dst = jnp.concatenate([edge_index[0], pad_idx]).reshape(-1, CHUNK)
    src = jnp.concatenate([edge_index[1], pad_idx]).reshape(-1, CHUNK)
    ew = jax.lax.bitcast_convert_type(
        jnp.pad(edge_weight, (0, N_PAD)), jnp.int32).reshape(-1, CHUNK)
    zpad = jnp.zeros((NW * CPW, 5, CHUNK), jnp.int32)
    meta = jnp.concatenate(
        [jnp.stack([src, dst, ew], axis=1), zpad], axis=1)
    partials = _sc_spmm(support, meta)
    return _combine(partials.reshape(NC, N_NODES, D))


# speedup vs baseline: 1.1798x; 1.0422x over previous
import dataclasses

import jax
import jax.numpy as jnp
from jax import lax
from jax.experimental import pallas as pl
from jax.experimental.pallas import tpu as pltpu
from jax.experimental.pallas import tpu_sc as plsc

N_NODES = 10000
N_EDGES = 320000
D = 128

NC = 2
NS = 16
NW = NC * NS
CHUNK = 112
CPW = 90
N_PAD = NW * CPW * CHUNK - N_EDGES

ROWS_PER_SUB = 624
TAIL_ROWS = N_NODES - NS * ROWS_PER_SUB
ZROWS = 104


def _matmul_body(x_ref, w_ref, o_ref):
    o_ref[...] = jnp.dot(x_ref[...], w_ref[...],
                         preferred_element_type=jnp.float32)


def _matmul(x, W):
    blk = 1000
    return pl.pallas_call(
        _matmul_body,
        grid=(N_NODES // blk,),
        in_specs=[
            pl.BlockSpec((blk, D), lambda i: (i, 0)),
            pl.BlockSpec((D, D), lambda i: (0, 0)),
        ],
        out_specs=pl.BlockSpec((blk, D), lambda i: (i, 0)),
        out_shape=jax.ShapeDtypeStruct((N_NODES, D), jnp.float32),
    )(x, W)


def _combine_body(p_ref, o_ref):
    o_ref[...] = jnp.maximum(p_ref[0] + p_ref[1], 0.0)


def _combine(partials):
    blk = 1000
    return pl.pallas_call(
        _combine_body,
        grid=(N_NODES // blk,),
        in_specs=[pl.BlockSpec((2, blk, D), lambda i: (0, i, 0))],
        out_specs=pl.BlockSpec((blk, D), lambda i: (i, 0)),
        out_shape=jax.ShapeDtypeStruct((N_NODES, D), jnp.float32),
    )(partials)


def _sc_body(support_hbm, meta_hbm, out_hbm,
             acc_spmem,
             meta0, meta1, meta2, meta3, meta4, meta5,
             semM0, semM1, semM2, semM3, semM4, semM5,
             rows0, rows1, rows2,
             semG0, semG1, semG2,
             semS0, semS1, semS2):
    core = lax.axis_index("c")
    sub = lax.axis_index("s")
    wid = sub * NC + core
    metas = (meta0, meta1, meta2, meta3, meta4, meta5)
    semMs = (semM0, semM1, semM2, semM3, semM4, semM5)
    rowss = (rows0, rows1, rows2)
    semGs = (semG0, semG1, semG2)
    semSs = (semS0, semS1, semS2)
    rowsA = rows0

    def issue_meta(n, ms):
        pltpu.async_copy(meta_hbm.at[wid + n * NW], metas[ms], semMs[ms])

    def wait_meta(n, ms):
        pltpu.make_async_copy(meta_hbm.at[wid + n * NW], metas[ms],
                              semMs[ms]).wait()

    for n in range(3):
        issue_meta(n, n)

    @pl.loop(0, ZROWS)
    def _(r):
        for g in range(D // 16):
            rowsA[r, pl.ds(g * 16, 16)] = jnp.zeros((16,), jnp.float32)

    base = sub * ROWS_PER_SUB
    for k in range(ROWS_PER_SUB // ZROWS):
        pltpu.sync_copy(rowsA.at[pl.ds(0, ZROWS)],
                        acc_spmem.at[pl.ds(base + k * ZROWS, ZROWS)])

    @pl.when(sub == NS - 1)
    def _():
        pltpu.sync_copy(rowsA.at[pl.ds(0, TAIL_ROWS)],
                        acc_spmem.at[pl.ds(NS * ROWS_PER_SUB, TAIL_ROWS)])

    plsc.subcore_barrier()

    def issue_gather(ms, rs):
        pltpu.async_copy(support_hbm.at[metas[ms].at[0]], rowss[rs],
                         semGs[rs])

    def process(ms, rs):
        meta_v, rows = metas[ms], rowss[rs]
        pltpu.make_async_copy(support_hbm.at[meta_v.at[0]], rows,
                              semGs[rs]).wait()

        @plsc.parallel_loop(0, CHUNK, step=16, unroll=2)
        def _(eg):
            w16 = plsc.bitcast(meta_v[2, pl.ds(eg, 16)], jnp.float32)
            for j in range(16):
                bw = jnp.full((16,), w16[j], jnp.float32)
                for g in range(D // 16):
                    sl = pl.ds(g * 16, 16)
                    rows[eg + j, sl] = rows[eg + j, sl] * bw

        pltpu.async_copy(rows, acc_spmem.at[meta_v.at[1]], semSs[rs],
                         add=True)

    def wait_scatter(ms, rs):
        pltpu.make_async_copy(rowss[rs], acc_spmem.at[metas[ms].at[1]],
                              semSs[rs]).wait()

    wait_meta(0, 0)
    issue_gather(0, 0)
    wait_meta(1, 1)
    issue_gather(1, 1)

    @pl.loop(0, CPW, step=6)
    def _(k):
        for b in range(6):
            n = k + b
            process(b, b % 3)
            if b == 0:
                @pl.when(k > 0)
                def _():
                    wait_scatter((b + 5) % 6, (b + 2) % 3)
            else:
                wait_scatter((b + 5) % 6, (b + 2) % 3)

            @pl.when(n + 3 < CPW)
            def _():
                issue_meta(n + 3, (b + 3) % 6)

            @pl.when(n + 2 < CPW)
            def _():
                wait_meta(n + 2, (b + 2) % 6)
                issue_gather((b + 2) % 6, (b + 2) % 3)

    wait_scatter((CPW - 1) % 6, (CPW - 1) % 3)

    plsc.subcore_barrier()

    ob = core * N_NODES + base
    for k in range(ROWS_PER_SUB // ZROWS):
        pltpu.sync_copy(acc_spmem.at[pl.ds(base + k * ZROWS, ZROWS)],
                        out_hbm.at[pl.ds(ob + k * ZROWS, ZROWS)])

    @pl.when(sub == NS - 1)
    def _():
        pltpu.sync_copy(acc_spmem.at[pl.ds(NS * ROWS_PER_SUB, TAIL_ROWS)],
                        out_hbm.at[pl.ds(core * N_NODES + NS * ROWS_PER_SUB,
                                         TAIL_ROWS)])


def _sc_spmm(support, meta):
    mesh = plsc.VectorSubcoreMesh(core_axis_name="c", subcore_axis_name="s")
    cp = pltpu.CompilerParams()
    if "needs_layout_passes" in pltpu.CompilerParams.__dataclass_fields__:
        cp = dataclasses.replace(cp, needs_layout_passes=False)
    f = pl.kernel(
        _sc_body,
        compiler_params=cp,
        out_type=jax.ShapeDtypeStruct((NC * N_NODES, D), jnp.float32),
        mesh=mesh,
        scratch_types=[
            pltpu.VMEM_SHARED((N_NODES, D), jnp.float32),
        ] + [pltpu.VMEM((8, CHUNK), jnp.int32) for _ in range(6)]
          + [pltpu.SemaphoreType.DMA for _ in range(6)]
          + [pltpu.VMEM((CHUNK, D), jnp.float32) for _ in range(3)]
          + [pltpu.SemaphoreType.DMA for _ in range(6)],
    )
    return f(support, meta)


def kernel(x, edge_index, edge_weight, W):
    support = _matmul(x, W)
    pad_idx = jnp.arange(N_PAD, dtype=jnp.int32) % N_NODES
    dst = jnp.concatenate([edge_index[0], pad_idx]).reshape(-1, CHUNK)
    src = jnp.concatenate([edge_index[1], pad_idx]).reshape(-1, CHUNK)
    ew = jax.lax.bitcast_convert_type(
        jnp.pad(edge_weight, (0, N_PAD)), jnp.int32).reshape(-1, CHUNK)
    zpad = jnp.zeros((NW * CPW, 5, CHUNK), jnp.int32)
    meta = jnp.concatenate(
        [jnp.stack([src, dst, ew], axis=1), zpad], axis=1)
    partials = _sc_spmm(support, meta)
    return _combine(partials.reshape(NC, N_NODES, D))
